# baseline (speedup 1.0000x reference)
import functools

import jax
import jax.numpy as jnp
from jax import lax
from jax.experimental import pallas as pl
from jax.experimental.pallas import tpu as pltpu
from jax.experimental.pallas import tpu_sc as plsc

NC = 2
NS = 16
NW = NC * NS
CH = 128


def _fill_const(ref, n, val):
    def body(k, _):
        ref[pl.ds(k * 16, 16)] = jnp.full((16,), val, jnp.float32)
        return 0
    lax.fori_loop(0, n // 16, body, 0)


def _mesh():
    return plsc.VectorSubcoreMesh(core_axis_name="c", subcore_axis_name="s",
                                  num_cores=NC, num_subcores=NS)


def _load_dst_chunks(ei_hbm, didx, sem, ebase, full):
    def body(j, _):
        pltpu.async_copy(ei_hbm.at[1, pl.ds(ebase + j * CH, CH)],
                         didx.at[j], sem)
        return 0
    lax.fori_loop(0, full, body, 0)


def _drain_dst_chunks(ei_hbm, didx, sem, ebase, full):
    def body(j, _):
        pltpu.make_async_copy(ei_hbm.at[1, pl.ds(ebase + j * CH, CH)],
                              didx.at[j], sem).wait()
        return 0
    lax.fori_loop(0, full, body, 0)


def _make_deg_kernel(npad, full, rem):
    rpt = npad // NS
    em = full * CH

    scratch = [
        pltpu.VMEM((full, CH), jnp.int32),
        pltpu.VMEM((CH,), jnp.float32),
        pltpu.VMEM((rpt,), jnp.float32),
        pltpu.SemaphoreType.DMA,
    ]
    if rem:
        scratch += [pltpu.VMEM((rem,), jnp.int32),
                    pltpu.VMEM((rem,), jnp.float32)]

    @functools.partial(
        pl.kernel,
        out_type=[jax.ShapeDtypeStruct((npad,), jnp.float32)] * NC,
        mesh=_mesh(),
        scratch_types=scratch + [pltpu.VMEM_SHARED((npad,), jnp.float32)],
    )
    def deg_k(ei_hbm, out0_hbm, out1_hbm, didx, ones_v, zero_v, sem, *rest):
        acc = rest[-1]
        cid = lax.axis_index("c")
        sid = lax.axis_index("s")
        wid = sid * NC + cid
        ebase = pl.multiple_of(wid * em, 8)

        _load_dst_chunks(ei_hbm, didx, sem, ebase, full)
        _fill_const(ones_v, CH, 1.0)
        _fill_const(zero_v, rpt, 0.0)
        pltpu.sync_copy(zero_v, acc.at[pl.ds(sid * rpt, rpt)])
        plsc.subcore_barrier()
        _drain_dst_chunks(ei_hbm, didx, sem, ebase, full)

        def body(j, _):
            pltpu.async_copy(ones_v, acc.at[didx.at[j]], sem, add=True)
            return 0
        lax.fori_loop(0, full, body, 0)

        def drain(j, _):
            pltpu.make_async_copy(ones_v, acc.at[didx.at[j]], sem).wait()
            return 0
        lax.fori_loop(0, full, drain, 0)

        if rem:
            ridx, rones = rest[0], rest[1]
            rbase = pl.multiple_of(NW * em + wid * rem, 8)
            _fill_const(rones, rem, 1.0)
            pltpu.sync_copy(ei_hbm.at[1, pl.ds(rbase, rem)], ridx)
            pltpu.sync_copy(rones, acc.at[ridx], add=True)

        plsc.subcore_barrier()
        pltpu.sync_copy(acc.at[pl.ds(sid * rpt, rpt)], zero_v)

        @pl.when(cid == 0)
        def _():
            pltpu.sync_copy(zero_v, out0_hbm.at[pl.ds(sid * rpt, rpt)])

        @pl.when(cid == 1)
        def _():
            pltpu.sync_copy(zero_v, out1_hbm.at[pl.ds(sid * rpt, rpt)])

    return deg_k


def _make_aggr_kernel(c, npad, full, rem):
    rpt = npad // NS
    em = full * CH
    assert full % 2 == 0 and full >= 4

    scratch = [
        pltpu.VMEM((full, CH), jnp.int32),
        pltpu.VMEM((CH,), jnp.int32),
        pltpu.VMEM((CH,), jnp.int32),
        pltpu.VMEM((CH, c), jnp.float32),
        pltpu.VMEM((CH, c), jnp.float32),
        pltpu.VMEM((16, c), jnp.float32),
        pltpu.SemaphoreType.DMA,
        pltpu.SemaphoreType.DMA,
        pltpu.SemaphoreType.DMA,
        pltpu.SemaphoreType.DMA,
        pltpu.SemaphoreType.DMA,
        pltpu.SemaphoreType.DMA,
    ]
    if rem:
        scratch += [pltpu.VMEM((rem,), jnp.int32),
                    pltpu.VMEM((rem, c), jnp.float32)]

    @functools.partial(
        pl.kernel,
        out_type=[jax.ShapeDtypeStruct((npad, c), jnp.float32)] * NC,
        mesh=_mesh(),
        scratch_types=scratch + [pltpu.VMEM_SHARED((npad, c), jnp.float32)],
    )
    def aggr_k(h2_hbm, ei_hbm, out0_hbm, out1_hbm,
               didx, si0, si1, gb0, gb1, zbuf,
               gsem0, gsem1, isem0, isem1, ssem0, ssem1, *rest):
        acc = rest[-1]
        cid = lax.axis_index("c")
        sid = lax.axis_index("s")
        wid = sid * NC + cid
        ebase = pl.multiple_of(wid * em, 8)

        _load_dst_chunks(ei_hbm, didx, isem0, ebase, full)

        def zfill(k, _):
            zbuf[k // 8, pl.ds((k % 8) * 16, 16)] = jnp.zeros((16,), jnp.float32)
            return 0
        lax.fori_loop(0, 16 * (c // 16), zfill, 0)

        def zcopy(k, _):
            pltpu.sync_copy(zbuf, acc.at[pl.ds(sid * rpt + k * 16, 16)])
            return 0
        lax.fori_loop(0, rpt // 16, zcopy, 0)
        plsc.subcore_barrier()
        _drain_dst_chunks(ei_hbm, didx, isem0, ebase, full)

        def iload(j, buf, sem):
            pltpu.async_copy(ei_hbm.at[0, pl.ds(ebase + j * CH, CH)], buf, sem)

        def iwait(buf, sem):
            pltpu.make_async_copy(
                ei_hbm.at[0, pl.ds(ebase, CH)], buf, sem).wait()

        pltpu.sync_copy(ei_hbm.at[0, pl.ds(ebase, CH)], si0)
        iload(1, si1, isem1)
        pltpu.async_copy(h2_hbm.at[si0], gb0, gsem0)

        def outer(g, _):
            j0 = 2 * g
            iwait(si1, isem1)
            pltpu.async_copy(h2_hbm.at[si1], gb1, gsem1)
            pltpu.make_async_copy(h2_hbm.at[si0], gb0, gsem0).wait()

            @pl.when(j0 + 2 < full)
            def _():
                iload(j0 + 2, si0, isem0)
            pltpu.sync_copy(gb0, acc.at[didx.at[j0]], add=True)

            @pl.when(j0 + 2 < full)
            def _():
                iwait(si0, isem0)
                pltpu.async_copy(h2_hbm.at[si0], gb0, gsem0)
            pltpu.make_async_copy(h2_hbm.at[si1], gb1, gsem1).wait()

            @pl.when(j0 + 3 < full)
            def _():
                iload(j0 + 3, si1, isem1)
            pltpu.sync_copy(gb1, acc.at[didx.at[j0 + 1]], add=True)
            return 0
        lax.fori_loop(0, full // 2, outer, 0)

        if rem:
            didxr, grem = rest[0], rest[1]
            rbase = pl.multiple_of(NW * em + wid * rem, 8)
            pltpu.sync_copy(ei_hbm.at[1, pl.ds(rbase, rem)], didxr)
            pltpu.sync_copy(
                ei_hbm.at[0, pl.ds(rbase, rem)], si0.at[pl.ds(0, rem)])
            pltpu.async_copy(h2_hbm.at[si0.at[pl.ds(0, rem)]], grem, gsem0).wait()
            pltpu.sync_copy(grem, acc.at[didxr], add=True)

        plsc.subcore_barrier()
        def dump(k, _):
            r = sid * rpt + k * 16
            pltpu.sync_copy(acc.at[pl.ds(r, 16)], zbuf)

            @pl.when(cid == 0)
            def _():
                pltpu.sync_copy(zbuf, out0_hbm.at[pl.ds(r, 16)])

            @pl.when(cid == 1)
            def _():
                pltpu.sync_copy(zbuf, out1_hbm.at[pl.ds(r, 16)])
            return 0
        lax.fori_loop(0, rpt // 16, dump, 0)

    return aggr_k


def _dis(d0, d1):
    deg = d0 + d1
    return jnp.where(deg > 0.0, lax.rsqrt(jnp.maximum(deg, 1.0)), 0.0)


def _scale_body(x_ref, d0_ref, d1_ref, o_ref):
    dis = _dis(d0_ref[...], d1_ref[...])
    o_ref[...] = x_ref[...] * dis.reshape(-1, 1)


def _fin_body(p0_ref, p1_ref, d0_ref, d1_ref, w_ref, b_ref, o_ref):
    dis = _dis(d0_ref[...], d1_ref[...])
    s = (p0_ref[...] + p1_ref[...]) * dis.reshape(-1, 1)
    h = jnp.dot(s, w_ref[...], preferred_element_type=jnp.float32)
    o_ref[...] = jnp.maximum(h + b_ref[...], 0.0)


def kernel(x, edge_index_K, W, b):
    n, c_in = x.shape
    c_out = W.shape[1]
    e = edge_index_K.shape[1]

    npad = ((n + 1 + 1023) // 1024) * 1024
    epad = ((e + NW * 8 - 1) // (NW * 8)) * (NW * 8)
    epw = epad // NW
    full = epw // CH
    rem = epw % CH

    ei = edge_index_K
    if epad > e:
        pad = epad - e
        ei = jnp.concatenate([
            edge_index_K,
            jnp.stack([jnp.zeros((pad,), jnp.int32),
                       n + (jnp.arange(pad, dtype=jnp.int32) % (npad - n))]),
        ], axis=1)

    d0, d1 = _make_deg_kernel(npad, full, rem)(ei)

    br = 1024
    grid = ((n + br - 1) // br,)
    x2 = pl.pallas_call(
        _scale_body,
        grid=grid,
        in_specs=[
            pl.BlockSpec((br, c_in), lambda i: (i, 0)),
            pl.BlockSpec((br,), lambda i: (i,)),
            pl.BlockSpec((br,), lambda i: (i,)),
        ],
        out_specs=pl.BlockSpec((br, c_in), lambda i: (i, 0)),
        out_shape=jax.ShapeDtypeStruct((n, c_in), jnp.float32),
    )(x, d0, d1)

    p0, p1 = _make_aggr_kernel(c_in, npad, full, rem)(x2, ei)

    out = pl.pallas_call(
        _fin_body,
        grid=grid,
        in_specs=[
            pl.BlockSpec((br, c_in), lambda i: (i, 0)),
            pl.BlockSpec((br, c_in), lambda i: (i, 0)),
            pl.BlockSpec((br,), lambda i: (i,)),
            pl.BlockSpec((br,), lambda i: (i,)),
            pl.BlockSpec((c_in, c_out), lambda i: (0, 0)),
            pl.BlockSpec((1, c_out), lambda i: (0, 0)),
        ],
        out_specs=pl.BlockSpec((br, c_out), lambda i: (i, 0)),
        out_shape=jax.ShapeDtypeStruct((n, c_out), jnp.float32),
    )(p0, p1, d0, d1, W, b.reshape(1, c_out))
    return out


# speedup vs baseline: 1.0660x; 1.0660x over previous
import functools

import jax
import jax.numpy as jnp
from jax import lax
from jax.experimental import pallas as pl
from jax.experimental.pallas import tpu as pltpu
from jax.experimental.pallas import tpu_sc as plsc

NC = 2
NS = 16
NW = NC * NS
CH = 128


def _fill_const(ref, n, val):
    def body(k, _):
        ref[pl.ds(k * 16, 16)] = jnp.full((16,), val, jnp.float32)
        return 0
    lax.fori_loop(0, n // 16, body, 0)


def _mesh():
    return plsc.VectorSubcoreMesh(core_axis_name="c", subcore_axis_name="s",
                                  num_cores=NC, num_subcores=NS)


def _load_dst_chunks(ei_hbm, didx, sem, ebase, full):
    def body(j, _):
        pltpu.async_copy(ei_hbm.at[1, pl.ds(ebase + j * CH, CH)],
                         didx.at[j], sem)
        return 0
    lax.fori_loop(0, full, body, 0)


def _drain_dst_chunks(ei_hbm, didx, sem, ebase, full):
    def body(j, _):
        pltpu.make_async_copy(ei_hbm.at[1, pl.ds(ebase + j * CH, CH)],
                              didx.at[j], sem).wait()
        return 0
    lax.fori_loop(0, full, body, 0)


def _make_deg_kernel(npad, full, rem):
    rpt = npad // NS
    em = full * CH

    scratch = [
        pltpu.VMEM((full, CH), jnp.int32),
        pltpu.VMEM((CH,), jnp.float32),
        pltpu.VMEM((rpt,), jnp.float32),
        pltpu.SemaphoreType.DMA,
    ]
    if rem:
        scratch += [pltpu.VMEM((rem,), jnp.int32),
                    pltpu.VMEM((rem,), jnp.float32)]

    @functools.partial(
        pl.kernel,
        out_type=[jax.ShapeDtypeStruct((npad,), jnp.float32)] * NC,
        mesh=_mesh(),
        scratch_types=scratch + [pltpu.VMEM_SHARED((npad,), jnp.float32)],
    )
    def deg_k(ei_hbm, out0_hbm, out1_hbm, didx, ones_v, zero_v, sem, *rest):
        acc = rest[-1]
        cid = lax.axis_index("c")
        sid = lax.axis_index("s")
        wid = sid * NC + cid
        ebase = pl.multiple_of(wid * em, 8)

        _load_dst_chunks(ei_hbm, didx, sem, ebase, full)
        _fill_const(ones_v, CH, 1.0)
        _fill_const(zero_v, rpt, 0.0)
        pltpu.sync_copy(zero_v, acc.at[pl.ds(sid * rpt, rpt)])
        plsc.subcore_barrier()
        _drain_dst_chunks(ei_hbm, didx, sem, ebase, full)

        def body(j, _):
            pltpu.async_copy(ones_v, acc.at[didx.at[j]], sem, add=True)
            return 0
        lax.fori_loop(0, full, body, 0)

        def drain(j, _):
            pltpu.make_async_copy(ones_v, acc.at[didx.at[j]], sem).wait()
            return 0
        lax.fori_loop(0, full, drain, 0)

        if rem:
            ridx, rones = rest[0], rest[1]
            rbase = pl.multiple_of(NW * em + wid * rem, 8)
            _fill_const(rones, rem, 1.0)
            pltpu.sync_copy(ei_hbm.at[1, pl.ds(rbase, rem)], ridx)
            pltpu.sync_copy(rones, acc.at[ridx], add=True)

        plsc.subcore_barrier()
        pltpu.sync_copy(acc.at[pl.ds(sid * rpt, rpt)], zero_v)

        @pl.when(cid == 0)
        def _():
            pltpu.sync_copy(zero_v, out0_hbm.at[pl.ds(sid * rpt, rpt)])

        @pl.when(cid == 1)
        def _():
            pltpu.sync_copy(zero_v, out1_hbm.at[pl.ds(sid * rpt, rpt)])

    return deg_k


def _make_aggr_kernel(c, npad, full, rem):
    rpt = npad // NS
    em = full * CH
    assert full % 2 == 0 and full >= 4

    scratch = [
        pltpu.VMEM((full, CH), jnp.int32),
        pltpu.VMEM((CH,), jnp.int32),
        pltpu.VMEM((CH,), jnp.int32),
        pltpu.VMEM((CH, c), jnp.float32),
        pltpu.VMEM((CH, c), jnp.float32),
        pltpu.VMEM((16, c), jnp.float32),
        pltpu.SemaphoreType.DMA,
        pltpu.SemaphoreType.DMA,
        pltpu.SemaphoreType.DMA,
        pltpu.SemaphoreType.DMA,
        pltpu.SemaphoreType.DMA,
        pltpu.SemaphoreType.DMA,
    ]
    if rem:
        scratch += [pltpu.VMEM((rem,), jnp.int32),
                    pltpu.VMEM((rem, c), jnp.float32)]

    @functools.partial(
        pl.kernel,
        out_type=[jax.ShapeDtypeStruct((npad, c), jnp.float32)] * NC,
        mesh=_mesh(),
        scratch_types=scratch + [pltpu.VMEM_SHARED((npad, c), jnp.float32)],
    )
    def aggr_k(h2_hbm, ei_hbm, out0_hbm, out1_hbm,
               didx, si0, si1, gb0, gb1, zbuf,
               gsem0, gsem1, isem0, isem1, ssem0, ssem1, *rest):
        acc = rest[-1]
        cid = lax.axis_index("c")
        sid = lax.axis_index("s")
        wid = sid * NC + cid
        ebase = pl.multiple_of(wid * em, 8)

        _load_dst_chunks(ei_hbm, didx, isem0, ebase, full)

        def zfill(k, _):
            zbuf[k // 8, pl.ds((k % 8) * 16, 16)] = jnp.zeros((16,), jnp.float32)
            return 0
        lax.fori_loop(0, 16 * (c // 16), zfill, 0)

        def zcopy(k, _):
            pltpu.async_copy(zbuf, acc.at[pl.ds(sid * rpt + k * 16, 16)], gsem0)
            return 0
        lax.fori_loop(0, rpt // 16, zcopy, 0)

        def zdrain(k, _):
            pltpu.make_async_copy(
                zbuf, acc.at[pl.ds(sid * rpt + k * 16, 16)], gsem0).wait()
            return 0
        lax.fori_loop(0, rpt // 16, zdrain, 0)
        plsc.subcore_barrier()
        _drain_dst_chunks(ei_hbm, didx, isem0, ebase, full)

        def iload(j, buf, sem):
            pltpu.async_copy(ei_hbm.at[0, pl.ds(ebase + j * CH, CH)], buf, sem)

        def iwait(buf, sem):
            pltpu.make_async_copy(
                ei_hbm.at[0, pl.ds(ebase, CH)], buf, sem).wait()

        pltpu.sync_copy(ei_hbm.at[0, pl.ds(ebase, CH)], si0)
        iload(1, si1, isem1)
        pltpu.async_copy(h2_hbm.at[si0], gb0, gsem0)

        def outer(g, _):
            j0 = 2 * g
            iwait(si1, isem1)
            pltpu.async_copy(h2_hbm.at[si1], gb1, gsem1)
            pltpu.make_async_copy(h2_hbm.at[si0], gb0, gsem0).wait()

            @pl.when(j0 + 2 < full)
            def _():
                iload(j0 + 2, si0, isem0)
            pltpu.sync_copy(gb0, acc.at[didx.at[j0]], add=True)

            @pl.when(j0 + 2 < full)
            def _():
                iwait(si0, isem0)
                pltpu.async_copy(h2_hbm.at[si0], gb0, gsem0)
            pltpu.make_async_copy(h2_hbm.at[si1], gb1, gsem1).wait()

            @pl.when(j0 + 3 < full)
            def _():
                iload(j0 + 3, si1, isem1)
            pltpu.sync_copy(gb1, acc.at[didx.at[j0 + 1]], add=True)
            return 0
        lax.fori_loop(0, full // 2, outer, 0)

        if rem:
            didxr, grem = rest[0], rest[1]
            rbase = pl.multiple_of(NW * em + wid * rem, 8)
            pltpu.sync_copy(ei_hbm.at[1, pl.ds(rbase, rem)], didxr)
            pltpu.sync_copy(
                ei_hbm.at[0, pl.ds(rbase, rem)], si0.at[pl.ds(0, rem)])
            pltpu.async_copy(h2_hbm.at[si0.at[pl.ds(0, rem)]], grem, gsem0).wait()
            pltpu.sync_copy(grem, acc.at[didxr], add=True)

        plsc.subcore_barrier()
        DR = 64
        assert rpt % DR == 0

        def hbm_write(r, gb, sem):
            @pl.when(cid == 0)
            def _():
                pltpu.async_copy(gb.at[pl.ds(0, DR)],
                                 out0_hbm.at[pl.ds(r, DR)], sem)

            @pl.when(cid == 1)
            def _():
                pltpu.async_copy(gb.at[pl.ds(0, DR)],
                                 out1_hbm.at[pl.ds(r, DR)], sem)

        def hbm_drain(r, gb, sem):
            @pl.when(cid == 0)
            def _():
                pltpu.make_async_copy(gb.at[pl.ds(0, DR)],
                                      out0_hbm.at[pl.ds(r, DR)], sem).wait()

            @pl.when(cid == 1)
            def _():
                pltpu.make_async_copy(gb.at[pl.ds(0, DR)],
                                      out1_hbm.at[pl.ds(r, DR)], sem).wait()

        bufs = (gb0, gb1)
        sems = (gsem0, gsem1)
        for k in range(rpt // DR):
            r = sid * rpt + k * DR
            gb, sem = bufs[k % 2], sems[k % 2]
            if k >= 2:
                hbm_drain(sid * rpt + (k - 2) * DR, gb, sem)
            pltpu.sync_copy(acc.at[pl.ds(r, DR)], gb.at[pl.ds(0, DR)])
            hbm_write(r, gb, sem)
        for k in range(max(rpt // DR - 2, 0), rpt // DR):
            hbm_drain(sid * rpt + k * DR, bufs[k % 2], sems[k % 2])

    return aggr_k


def _dis(d0, d1):
    deg = d0 + d1
    return jnp.where(deg > 0.0, lax.rsqrt(jnp.maximum(deg, 1.0)), 0.0)


def _mm_body(x_ref, w_ref, d0_ref, d1_ref, o_ref):
    dis = _dis(d0_ref[...], d1_ref[...])
    h = jnp.dot(x_ref[...], w_ref[...], preferred_element_type=jnp.float32)
    o_ref[...] = h * dis.reshape(-1, 1)


def _fin_body(p0_ref, p1_ref, d0_ref, d1_ref, b_ref, o_ref):
    dis = _dis(d0_ref[...], d1_ref[...])
    s = (p0_ref[...] + p1_ref[...]) * dis.reshape(-1, 1) + b_ref[...]
    o_ref[...] = jnp.maximum(s, 0.0)


def kernel(x, edge_index_K, W, b):
    n, c_in = x.shape
    c_out = W.shape[1]
    e = edge_index_K.shape[1]

    npad = ((n + 1 + 1023) // 1024) * 1024
    epad = ((e + NW * 8 - 1) // (NW * 8)) * (NW * 8)
    epw = epad // NW
    full = epw // CH
    rem = epw % CH

    ei = edge_index_K
    if epad > e:
        pad = epad - e
        ei = jnp.concatenate([
            edge_index_K,
            jnp.stack([jnp.zeros((pad,), jnp.int32),
                       n + (jnp.arange(pad, dtype=jnp.int32) % (npad - n))]),
        ], axis=1)

    d0, d1 = _make_deg_kernel(npad, full, rem)(ei)

    br = 1024
    grid = ((n + br - 1) // br,)
    h2 = pl.pallas_call(
        _mm_body,
        grid=grid,
        in_specs=[
            pl.BlockSpec((br, c_in), lambda i: (i, 0)),
            pl.BlockSpec((c_in, c_out), lambda i: (0, 0)),
            pl.BlockSpec((br,), lambda i: (i,)),
            pl.BlockSpec((br,), lambda i: (i,)),
        ],
        out_specs=pl.BlockSpec((br, c_out), lambda i: (i, 0)),
        out_shape=jax.ShapeDtypeStruct((n, c_out), jnp.float32),
    )(x, W, d0, d1)

    p0, p1 = _make_aggr_kernel(c_out, npad, full, rem)(h2, ei)

    out = pl.pallas_call(
        _fin_body,
        grid=grid,
        in_specs=[
            pl.BlockSpec((br, c_out), lambda i: (i, 0)),
            pl.BlockSpec((br, c_out), lambda i: (i, 0)),
            pl.BlockSpec((br,), lambda i: (i,)),
            pl.BlockSpec((br,), lambda i: (i,)),
            pl.BlockSpec((1, c_out), lambda i: (0, 0)),
        ],
        out_specs=pl.BlockSpec((br, c_out), lambda i: (i, 0)),
        out_shape=jax.ShapeDtypeStruct((n, c_out), jnp.float32),
    )(p0, p1, d0, d1, b.reshape(1, c_out))
    return out


# speedup vs baseline: 1.0964x; 1.0286x over previous
import functools

import jax
import jax.numpy as jnp
from jax import lax
from jax.experimental import pallas as pl
from jax.experimental.pallas import tpu as pltpu
from jax.experimental.pallas import tpu_sc as plsc

NC = 2
NS = 16
NW = NC * NS
CH = 128


def _fill_const(ref, n, val):
    def body(k, _):
        ref[pl.ds(k * 16, 16)] = jnp.full((16,), val, jnp.float32)
        return 0
    lax.fori_loop(0, n // 16, body, 0)


def _mesh():
    return plsc.VectorSubcoreMesh(core_axis_name="c", subcore_axis_name="s",
                                  num_cores=NC, num_subcores=NS)


def _load_dst_chunks(ei_hbm, didx, sem, ebase, full):
    def body(j, _):
        pltpu.async_copy(ei_hbm.at[1, pl.ds(ebase + j * CH, CH)],
                         didx.at[j], sem)
        return 0
    lax.fori_loop(0, full, body, 0)


def _drain_dst_chunks(ei_hbm, didx, sem, ebase, full):
    def body(j, _):
        pltpu.make_async_copy(ei_hbm.at[1, pl.ds(ebase + j * CH, CH)],
                              didx.at[j], sem).wait()
        return 0
    lax.fori_loop(0, full, body, 0)


def _make_deg_kernel(npad, full, rem):
    rpt = npad // NS
    em = full * CH

    scratch = [
        pltpu.VMEM((full, CH), jnp.int32),
        pltpu.VMEM((CH,), jnp.float32),
        pltpu.VMEM((rpt,), jnp.float32),
        pltpu.SemaphoreType.DMA,
    ]
    if rem:
        scratch += [pltpu.VMEM((rem,), jnp.int32),
                    pltpu.VMEM((rem,), jnp.float32)]

    @functools.partial(
        pl.kernel,
        out_type=[jax.ShapeDtypeStruct((npad,), jnp.float32)] * NC,
        mesh=_mesh(),
        scratch_types=scratch + [pltpu.VMEM_SHARED((npad,), jnp.float32)],
    )
    def deg_k(ei_hbm, out0_hbm, out1_hbm, didx, ones_v, zero_v, sem, *rest):
        acc = rest[-1]
        cid = lax.axis_index("c")
        sid = lax.axis_index("s")
        wid = sid * NC + cid
        ebase = pl.multiple_of(wid * em, 8)

        _load_dst_chunks(ei_hbm, didx, sem, ebase, full)
        _fill_const(ones_v, CH, 1.0)
        _fill_const(zero_v, rpt, 0.0)
        pltpu.sync_copy(zero_v, acc.at[pl.ds(sid * rpt, rpt)])
        plsc.subcore_barrier()
        _drain_dst_chunks(ei_hbm, didx, sem, ebase, full)

        def body(j, _):
            pltpu.async_copy(ones_v, acc.at[didx.at[j]], sem, add=True)
            return 0
        lax.fori_loop(0, full, body, 0)

        def drain(j, _):
            pltpu.make_async_copy(ones_v, acc.at[didx.at[j]], sem).wait()
            return 0
        lax.fori_loop(0, full, drain, 0)

        if rem:
            ridx, rones = rest[0], rest[1]
            rbase = pl.multiple_of(NW * em + wid * rem, 8)
            _fill_const(rones, rem, 1.0)
            pltpu.sync_copy(ei_hbm.at[1, pl.ds(rbase, rem)], ridx)
            pltpu.sync_copy(rones, acc.at[ridx], add=True)

        plsc.subcore_barrier()
        pltpu.sync_copy(acc.at[pl.ds(sid * rpt, rpt)], zero_v)

        @pl.when(cid == 0)
        def _():
            pltpu.sync_copy(zero_v, out0_hbm.at[pl.ds(sid * rpt, rpt)])

        @pl.when(cid == 1)
        def _():
            pltpu.sync_copy(zero_v, out1_hbm.at[pl.ds(sid * rpt, rpt)])

    return deg_k


def _make_aggr_kernel(c, npad, full, rem):
    rpt = npad // NS
    em = full * CH
    assert full % 2 == 0 and full >= 4

    scratch = [
        pltpu.VMEM((full, CH), jnp.int32),
        pltpu.VMEM((CH,), jnp.int32),
        pltpu.VMEM((CH,), jnp.int32),
        pltpu.VMEM((CH, c), jnp.float32),
        pltpu.VMEM((CH, c), jnp.float32),
        pltpu.VMEM((16, c), jnp.float32),
        pltpu.SemaphoreType.DMA,
        pltpu.SemaphoreType.DMA,
        pltpu.SemaphoreType.DMA,
        pltpu.SemaphoreType.DMA,
        pltpu.SemaphoreType.DMA,
        pltpu.SemaphoreType.DMA,
    ]
    if rem:
        scratch += [pltpu.VMEM((rem,), jnp.int32),
                    pltpu.VMEM((rem, c), jnp.float32)]

    @functools.partial(
        pl.kernel,
        out_type=[jax.ShapeDtypeStruct((npad, c), jnp.float32)] * NC,
        mesh=_mesh(),
        scratch_types=scratch + [pltpu.VMEM_SHARED((npad, c), jnp.float32)],
    )
    def aggr_k(h2_hbm, ei_hbm, out0_hbm, out1_hbm,
               didx, si0, si1, gb0, gb1, zbuf,
               gsem0, gsem1, isem0, isem1, ssem0, ssem1, *rest):
        acc = rest[-1]
        cid = lax.axis_index("c")
        sid = lax.axis_index("s")
        wid = sid * NC + cid
        ebase = pl.multiple_of(wid * em, 8)

        _load_dst_chunks(ei_hbm, didx, isem0, ebase, full)

        def zfill(k, _):
            zbuf[k // 8, pl.ds((k % 8) * 16, 16)] = jnp.zeros((16,), jnp.float32)
            return 0
        lax.fori_loop(0, 16 * (c // 16), zfill, 0)

        def zcopy(k, _):
            pltpu.async_copy(zbuf, acc.at[pl.ds(sid * rpt + k * 16, 16)], gsem0)
            return 0
        lax.fori_loop(0, rpt // 16, zcopy, 0)

        def zdrain(k, _):
            pltpu.make_async_copy(
                zbuf, acc.at[pl.ds(sid * rpt + k * 16, 16)], gsem0).wait()
            return 0
        lax.fori_loop(0, rpt // 16, zdrain, 0)
        plsc.subcore_barrier()
        _drain_dst_chunks(ei_hbm, didx, isem0, ebase, full)

        def iload(j, buf, sem):
            pltpu.async_copy(ei_hbm.at[0, pl.ds(ebase + j * CH, CH)], buf, sem)

        def iwait(buf, sem):
            pltpu.make_async_copy(
                ei_hbm.at[0, pl.ds(ebase, CH)], buf, sem).wait()

        pltpu.sync_copy(ei_hbm.at[0, pl.ds(ebase, CH)], si0)
        iload(1, si1, isem1)
        pltpu.async_copy(h2_hbm.at[si0], gb0, gsem0)

        def outer(g, _):
            j0 = 2 * g
            iwait(si1, isem1)
            pltpu.async_copy(h2_hbm.at[si1], gb1, gsem1)
            pltpu.make_async_copy(h2_hbm.at[si0], gb0, gsem0).wait()

            @pl.when(j0 + 2 < full)
            def _():
                iload(j0 + 2, si0, isem0)
            pltpu.sync_copy(gb0, acc.at[didx.at[j0]], add=True)

            @pl.when(j0 + 2 < full)
            def _():
                iwait(si0, isem0)
                pltpu.async_copy(h2_hbm.at[si0], gb0, gsem0)
            pltpu.make_async_copy(h2_hbm.at[si1], gb1, gsem1).wait()

            @pl.when(j0 + 3 < full)
            def _():
                iload(j0 + 3, si1, isem1)
            pltpu.sync_copy(gb1, acc.at[didx.at[j0 + 1]], add=True)
            return 0
        lax.fori_loop(0, full // 2, outer, 0)

        if rem:
            didxr, grem = rest[0], rest[1]
            rbase = pl.multiple_of(NW * em + wid * rem, 8)
            pltpu.sync_copy(ei_hbm.at[1, pl.ds(rbase, rem)], didxr)
            pltpu.sync_copy(
                ei_hbm.at[0, pl.ds(rbase, rem)], si0.at[pl.ds(0, rem)])
            pltpu.async_copy(h2_hbm.at[si0.at[pl.ds(0, rem)]], grem, gsem0).wait()
            pltpu.sync_copy(grem, acc.at[didxr], add=True)

        plsc.subcore_barrier()
        DR = 64
        assert rpt % DR == 0

        def hbm_write(r, gb, sem):
            @pl.when(cid == 0)
            def _():
                pltpu.async_copy(gb.at[pl.ds(0, DR)],
                                 out0_hbm.at[pl.ds(r, DR)], sem)

            @pl.when(cid == 1)
            def _():
                pltpu.async_copy(gb.at[pl.ds(0, DR)],
                                 out1_hbm.at[pl.ds(r, DR)], sem)

        def hbm_drain(r, gb, sem):
            @pl.when(cid == 0)
            def _():
                pltpu.make_async_copy(gb.at[pl.ds(0, DR)],
                                      out0_hbm.at[pl.ds(r, DR)], sem).wait()

            @pl.when(cid == 1)
            def _():
                pltpu.make_async_copy(gb.at[pl.ds(0, DR)],
                                      out1_hbm.at[pl.ds(r, DR)], sem).wait()

        bufs = (gb0, gb1)
        sems = (gsem0, gsem1)
        for k in range(rpt // DR):
            r = sid * rpt + k * DR
            gb, sem = bufs[k % 2], sems[k % 2]
            if k >= 2:
                hbm_drain(sid * rpt + (k - 2) * DR, gb, sem)
            pltpu.sync_copy(acc.at[pl.ds(r, DR)], gb.at[pl.ds(0, DR)])
            hbm_write(r, gb, sem)
        for k in range(max(rpt // DR - 2, 0), rpt // DR):
            hbm_drain(sid * rpt + k * DR, bufs[k % 2], sems[k % 2])

    return aggr_k


def _dis(d0, d1):
    deg = d0 + d1
    return jnp.where(deg > 0.0, lax.rsqrt(jnp.maximum(deg, 1.0)), 0.0)


def _mm_body(x_ref, w_ref, d0_ref, d1_ref, o_ref):
    dis = _dis(d0_ref[...], d1_ref[...])
    h = jnp.dot(x_ref[...], w_ref[...], preferred_element_type=jnp.float32)
    o_ref[...] = h * dis.reshape(-1, 1)


def _fin_body(p0_ref, p1_ref, d0_ref, d1_ref, b_ref, o_ref):
    dis = _dis(d0_ref[...], d1_ref[...])
    s = (p0_ref[...] + p1_ref[...]) * dis.reshape(-1, 1) + b_ref[...]
    o_ref[...] = jnp.maximum(s, 0.0)


def kernel(x, edge_index_K, W, b):
    n, c_in = x.shape
    c_out = W.shape[1]
    e = edge_index_K.shape[1]

    npad = ((n + 1 + 1023) // 1024) * 1024
    epad = ((e + NW * 8 - 1) // (NW * 8)) * (NW * 8)
    epw = epad // NW
    full = epw // CH
    rem = epw % CH

    ei = edge_index_K
    if epad > e:
        pad = epad - e
        ei = jnp.concatenate([
            edge_index_K,
            jnp.stack([jnp.zeros((pad,), jnp.int32),
                       n + (jnp.arange(pad, dtype=jnp.int32) % (npad - n))]),
        ], axis=1)

    d0, d1 = _make_deg_kernel(npad, full, rem)(ei)

    br = 2048
    grid = ((n + br - 1) // br,)
    h2 = pl.pallas_call(
        _mm_body,
        grid=grid,
        in_specs=[
            pl.BlockSpec((br, c_in), lambda i: (i, 0)),
            pl.BlockSpec((c_in, c_out), lambda i: (0, 0)),
            pl.BlockSpec((br,), lambda i: (i,)),
            pl.BlockSpec((br,), lambda i: (i,)),
        ],
        out_specs=pl.BlockSpec((br, c_out), lambda i: (i, 0)),
        out_shape=jax.ShapeDtypeStruct((n, c_out), jnp.float32),
    )(x, W, d0, d1)

    p0, p1 = _make_aggr_kernel(c_out, npad, full, rem)(h2, ei)

    out = pl.pallas_call(
        _fin_body,
        grid=grid,
        in_specs=[
            pl.BlockSpec((br, c_out), lambda i: (i, 0)),
            pl.BlockSpec((br, c_out), lambda i: (i, 0)),
            pl.BlockSpec((br,), lambda i: (i,)),
            pl.BlockSpec((br,), lambda i: (i,)),
            pl.BlockSpec((1, c_out), lambda i: (0, 0)),
        ],
        out_specs=pl.BlockSpec((br, c_out), lambda i: (i, 0)),
        out_shape=jax.ShapeDtypeStruct((n, c_out), jnp.float32),
    )(p0, p1, d0, d1, b.reshape(1, c_out))
    return out


# speedup vs baseline: 1.1144x; 1.0163x over previous
import functools

import jax
import jax.numpy as jnp
from jax import lax
from jax.experimental import pallas as pl
from jax.experimental.pallas import tpu as pltpu
from jax.experimental.pallas import tpu_sc as plsc

NC = 2
NS = 16
NW = NC * NS
CH = 128


def _fill_const(ref, n, val):
    def body(k, _):
        ref[pl.ds(k * 16, 16)] = jnp.full((16,), val, jnp.float32)
        return 0
    lax.fori_loop(0, n // 16, body, 0)


def _mesh():
    return plsc.VectorSubcoreMesh(core_axis_name="c", subcore_axis_name="s",
                                  num_cores=NC, num_subcores=NS)


def _load_dst_chunks(ei_hbm, didx, sem, ebase, full):
    def body(j, _):
        pltpu.async_copy(ei_hbm.at[1, pl.ds(ebase + j * CH, CH)],
                         didx.at[j], sem)
        return 0
    lax.fori_loop(0, full, body, 0)


def _drain_dst_chunks(ei_hbm, didx, sem, ebase, full):
    def body(j, _):
        pltpu.make_async_copy(ei_hbm.at[1, pl.ds(ebase + j * CH, CH)],
                              didx.at[j], sem).wait()
        return 0
    lax.fori_loop(0, full, body, 0)


def _make_deg_kernel(npad, full, rem):
    rpt = npad // NS
    em = full * CH

    scratch = [
        pltpu.VMEM((full, CH), jnp.int32),
        pltpu.VMEM((CH,), jnp.float32),
        pltpu.VMEM((rpt,), jnp.float32),
        pltpu.SemaphoreType.DMA,
    ]
    if rem:
        scratch += [pltpu.VMEM((rem,), jnp.int32),
                    pltpu.VMEM((rem,), jnp.float32)]

    @functools.partial(
        pl.kernel,
        out_type=[jax.ShapeDtypeStruct((npad,), jnp.float32)] * NC,
        mesh=_mesh(),
        scratch_types=scratch + [pltpu.VMEM_SHARED((npad,), jnp.float32)],
    )
    def deg_k(ei_hbm, out0_hbm, out1_hbm, didx, ones_v, zero_v, sem, *rest):
        acc = rest[-1]
        cid = lax.axis_index("c")
        sid = lax.axis_index("s")
        wid = sid * NC + cid
        ebase = pl.multiple_of(wid * em, 8)

        _load_dst_chunks(ei_hbm, didx, sem, ebase, full)
        _fill_const(ones_v, CH, 1.0)
        _fill_const(zero_v, rpt, 0.0)
        pltpu.sync_copy(zero_v, acc.at[pl.ds(sid * rpt, rpt)])
        plsc.subcore_barrier()
        _drain_dst_chunks(ei_hbm, didx, sem, ebase, full)

        def body(j, _):
            pltpu.async_copy(ones_v, acc.at[didx.at[j]], sem, add=True)
            return 0
        lax.fori_loop(0, full, body, 0)

        def drain(j, _):
            pltpu.make_async_copy(ones_v, acc.at[didx.at[j]], sem).wait()
            return 0
        lax.fori_loop(0, full, drain, 0)

        if rem:
            ridx, rones = rest[0], rest[1]
            rbase = pl.multiple_of(NW * em + wid * rem, 8)
            _fill_const(rones, rem, 1.0)
            pltpu.sync_copy(ei_hbm.at[1, pl.ds(rbase, rem)], ridx)
            pltpu.sync_copy(rones, acc.at[ridx], add=True)

        plsc.subcore_barrier()
        pltpu.sync_copy(acc.at[pl.ds(sid * rpt, rpt)], zero_v)

        @pl.when(cid == 0)
        def _():
            pltpu.sync_copy(zero_v, out0_hbm.at[pl.ds(sid * rpt, rpt)])

        @pl.when(cid == 1)
        def _():
            pltpu.sync_copy(zero_v, out1_hbm.at[pl.ds(sid * rpt, rpt)])

    return deg_k


def _make_aggr_kernel(c, npad, full, rem):
    rpt = npad // NS
    em = full * CH
    assert full % 2 == 0 and full >= 4

    scratch = [
        pltpu.VMEM((full, CH), jnp.int32),
        pltpu.VMEM((CH,), jnp.int32),
        pltpu.VMEM((CH,), jnp.int32),
        pltpu.VMEM((CH, c), jnp.float32),
        pltpu.VMEM((CH, c), jnp.float32),
        pltpu.VMEM((16, c), jnp.float32),
        pltpu.SemaphoreType.DMA,
        pltpu.SemaphoreType.DMA,
        pltpu.SemaphoreType.DMA,
        pltpu.SemaphoreType.DMA,
        pltpu.SemaphoreType.DMA,
        pltpu.SemaphoreType.DMA,
    ]
    if rem:
        scratch += [pltpu.VMEM((rem,), jnp.int32),
                    pltpu.VMEM((rem, c), jnp.float32)]

    @functools.partial(
        pl.kernel,
        out_type=[jax.ShapeDtypeStruct((npad, c), jnp.float32)] * NC,
        mesh=_mesh(),
        scratch_types=scratch + [pltpu.VMEM_SHARED((npad, c), jnp.float32)],
    )
    def aggr_k(h2_hbm, ei_hbm, out0_hbm, out1_hbm,
               didx, si0, si1, gb0, gb1, zbuf,
               gsem0, gsem1, isem0, isem1, ssem0, ssem1, *rest):
        acc = rest[-1]
        cid = lax.axis_index("c")
        sid = lax.axis_index("s")
        wid = sid * NC + cid
        ebase = pl.multiple_of(wid * em, 8)

        _load_dst_chunks(ei_hbm, didx, isem0, ebase, full)

        def zfill(k, _):
            zbuf[k // 8, pl.ds((k % 8) * 16, 16)] = jnp.zeros((16,), jnp.float32)
            return 0
        lax.fori_loop(0, 16 * (c // 16), zfill, 0)

        def zcopy(k, _):
            pltpu.async_copy(zbuf, acc.at[pl.ds(sid * rpt + k * 16, 16)], gsem0)
            return 0
        lax.fori_loop(0, rpt // 16, zcopy, 0)

        def zdrain(k, _):
            pltpu.make_async_copy(
                zbuf, acc.at[pl.ds(sid * rpt + k * 16, 16)], gsem0).wait()
            return 0
        lax.fori_loop(0, rpt // 16, zdrain, 0)
        plsc.subcore_barrier()
        _drain_dst_chunks(ei_hbm, didx, isem0, ebase, full)

        def iload(j, buf, sem):
            pltpu.async_copy(ei_hbm.at[0, pl.ds(ebase + j * CH, CH)], buf, sem)

        def iwait(buf, sem):
            pltpu.make_async_copy(
                ei_hbm.at[0, pl.ds(ebase, CH)], buf, sem).wait()

        pltpu.sync_copy(ei_hbm.at[0, pl.ds(ebase, CH)], si0)
        iload(1, si1, isem1)
        pltpu.async_copy(h2_hbm.at[si0], gb0, gsem0)

        def outer(g, _):
            j0 = 2 * g
            iwait(si1, isem1)
            pltpu.async_copy(h2_hbm.at[si1], gb1, gsem1)
            pltpu.make_async_copy(h2_hbm.at[si0], gb0, gsem0).wait()

            @pl.when(j0 + 2 < full)
            def _():
                iload(j0 + 2, si0, isem0)
            pltpu.sync_copy(gb0, acc.at[didx.at[j0]], add=True)

            @pl.when(j0 + 2 < full)
            def _():
                iwait(si0, isem0)
                pltpu.async_copy(h2_hbm.at[si0], gb0, gsem0)
            pltpu.make_async_copy(h2_hbm.at[si1], gb1, gsem1).wait()

            @pl.when(j0 + 3 < full)
            def _():
                iload(j0 + 3, si1, isem1)
            pltpu.sync_copy(gb1, acc.at[didx.at[j0 + 1]], add=True)
            return 0
        lax.fori_loop(0, full // 2, outer, 0)

        if rem:
            didxr, grem = rest[0], rest[1]
            rbase = pl.multiple_of(NW * em + wid * rem, 8)
            pltpu.sync_copy(ei_hbm.at[1, pl.ds(rbase, rem)], didxr)
            pltpu.sync_copy(
                ei_hbm.at[0, pl.ds(rbase, rem)], si0.at[pl.ds(0, rem)])
            pltpu.async_copy(h2_hbm.at[si0.at[pl.ds(0, rem)]], grem, gsem0).wait()
            pltpu.sync_copy(grem, acc.at[didxr], add=True)

        plsc.subcore_barrier()
        DR = 64
        assert rpt % DR == 0

        def hbm_write(r, gb, sem):
            @pl.when(cid == 0)
            def _():
                pltpu.async_copy(gb.at[pl.ds(0, DR)],
                                 out0_hbm.at[pl.ds(r, DR)], sem)

            @pl.when(cid == 1)
            def _():
                pltpu.async_copy(gb.at[pl.ds(0, DR)],
                                 out1_hbm.at[pl.ds(r, DR)], sem)

        def hbm_drain(r, gb, sem):
            @pl.when(cid == 0)
            def _():
                pltpu.make_async_copy(gb.at[pl.ds(0, DR)],
                                      out0_hbm.at[pl.ds(r, DR)], sem).wait()

            @pl.when(cid == 1)
            def _():
                pltpu.make_async_copy(gb.at[pl.ds(0, DR)],
                                      out1_hbm.at[pl.ds(r, DR)], sem).wait()

        bufs = (gb0, gb1)
        sems = (gsem0, gsem1)
        for k in range(rpt // DR):
            r = sid * rpt + k * DR
            gb, sem = bufs[k % 2], sems[k % 2]
            if k >= 2:
                hbm_drain(sid * rpt + (k - 2) * DR, gb, sem)
            pltpu.sync_copy(acc.at[pl.ds(r, DR)], gb.at[pl.ds(0, DR)])
            hbm_write(r, gb, sem)
        for k in range(max(rpt // DR - 2, 0), rpt // DR):
            hbm_drain(sid * rpt + k * DR, bufs[k % 2], sems[k % 2])

    return aggr_k


def _dis(d0, d1):
    deg = d0 + d1
    return jnp.where(deg > 0.0, lax.rsqrt(jnp.maximum(deg, 1.0)), 0.0)


def _mm_body(x_ref, w_ref, d0_ref, d1_ref, o_ref):
    dis = _dis(d0_ref[...], d1_ref[...])
    h = jnp.dot(x_ref[...], w_ref[...], preferred_element_type=jnp.float32)
    o_ref[...] = h * dis.reshape(-1, 1)


def _fin_body(p0_ref, p1_ref, d0_ref, d1_ref, b_ref, o_ref):
    dis = _dis(d0_ref[...], d1_ref[...])
    s = (p0_ref[...] + p1_ref[...]) * dis.reshape(-1, 1) + b_ref[...]
    o_ref[...] = jnp.maximum(s, 0.0)


def kernel(x, edge_index_K, W, b):
    n, c_in = x.shape
    c_out = W.shape[1]
    e = edge_index_K.shape[1]

    npad = ((n + 1 + 1023) // 1024) * 1024
    epad = ((e + NW * 8 - 1) // (NW * 8)) * (NW * 8)
    epw = epad // NW
    full = epw // CH
    rem = epw % CH

    ei = edge_index_K
    if epad > e:
        pad = epad - e
        ei = jnp.concatenate([
            edge_index_K,
            jnp.stack([jnp.zeros((pad,), jnp.int32),
                       n + (jnp.arange(pad, dtype=jnp.int32) % (npad - n))]),
        ], axis=1)

    d0, d1 = _make_deg_kernel(npad, full, rem)(ei)

    br = 4096
    grid = ((n + br - 1) // br,)
    h2 = pl.pallas_call(
        _mm_body,
        grid=grid,
        in_specs=[
            pl.BlockSpec((br, c_in), lambda i: (i, 0)),
            pl.BlockSpec((c_in, c_out), lambda i: (0, 0)),
            pl.BlockSpec((br,), lambda i: (i,)),
            pl.BlockSpec((br,), lambda i: (i,)),
        ],
        out_specs=pl.BlockSpec((br, c_out), lambda i: (i, 0)),
        out_shape=jax.ShapeDtypeStruct((n, c_out), jnp.float32),
    )(x, W, d0, d1)

    p0, p1 = _make_aggr_kernel(c_out, npad, full, rem)(h2, ei)

    out = pl.pallas_call(
        _fin_body,
        grid=grid,
        in_specs=[
            pl.BlockSpec((br, c_out), lambda i: (i, 0)),
            pl.BlockSpec((br, c_out), lambda i: (i, 0)),
            pl.BlockSpec((br,), lambda i: (i,)),
            pl.BlockSpec((br,), lambda i: (i,)),
            pl.BlockSpec((1, c_out), lambda i: (0, 0)),
        ],
        out_specs=pl.BlockSpec((br, c_out), lambda i: (i, 0)),
        out_shape=jax.ShapeDtypeStruct((n, c_out), jnp.float32),
    )(p0, p1, d0, d1, b.reshape(1, c_out))
    return out
